# Initial kernel scaffold; baseline (speedup 1.0000x reference)
import jax
import jax.numpy as jnp
from jax.experimental import pallas as pl


def kernel(x, edge_index, edge_type, pred_weight, W1, b1, W2, b2, W3, b3, W4, b4, g1, be1, g2, be2, g3, be3):
    raise NotImplementedError("write your pallas kernel here")


# speedup vs baseline: 10.3761x; 10.3761x over previous
import functools

import jax
import jax.numpy as jnp
from jax import lax
from jax.experimental import pallas as pl
from jax.experimental.pallas import tpu as pltpu
import jax.experimental.pallas.tpu_sc as plsc

N = 10000
E = 160000
R = 6
EPS = 1e-5
SLOPE = 0.01

NC = 2
NS = 16
ET = E // NS
KC = 80
NCH = ET // KC
RPT = N // NS
CNT_PAD = 61440
ZPT = CNT_PAD // NS
QW = ZPT // 8

_MESH = plsc.VectorSubcoreMesh(
    core_axis_name="c", subcore_axis_name="s", num_cores=NC, num_subcores=NS)


def _edge_weight_body(et2, src2, dst2, pw2, w2o, gao, cnto,
                      etv, srcv, dstv, pwv, cntv, gab, wb, redv, tmpv,
                      spc):
  c = lax.axis_index("c")
  t = lax.axis_index("s")

  @pl.when(c == 0)
  def _():
    pltpu.sync_copy(et2.at[t], etv)
    pltpu.sync_copy(src2.at[t], srcv)
    pltpu.sync_copy(dst2.at[t], dstv)
    pltpu.sync_copy(pw2.at[t], pwv)

    def zero_j(j, carry):
      cntv[pl.ds(j * 16, 16)] = jnp.zeros((16,), jnp.float32)
      return carry
    lax.fori_loop(0, CNT_PAD // 16, zero_j, 0)

    for gc in range(5):
      def chunk_j(j, carry):
        off = gc * 2000 + j * 16
        et16 = etv[pl.ds(off, 16)]
        d16 = dstv[pl.ds(off, 16)]
        s16 = srcv[pl.ds(off, 16)]
        cidx = et16 * N + d16
        plsc.addupdate_scatter(cntv, [cidx], jnp.ones((16,), jnp.float32))
        gab[pl.ds(j * 16, 16)] = et16 * N + s16
        return carry
      lax.fori_loop(0, 125, chunk_j, 0)
      pltpu.sync_copy(gab, gao.at[t, pl.ds(gc * 2000, 2000)])

    qoff = (t % 8) * QW
    rg_local = t // 8
    for b in range(8):
      for rb in range(2):
        pltpu.sync_copy(cntv.at[pl.ds((2 * b + rb) * ZPT, ZPT)],
                        spc.at[t, rb])
      plsc.subcore_barrier()
      pltpu.sync_copy(spc.at[0, rg_local, pl.ds(qoff, QW)], redv)
      for i in range(1, NS):
        pltpu.sync_copy(spc.at[i, rg_local, pl.ds(qoff, QW)], tmpv)
        def add_j(j, carry):
          s = pl.ds(j * 16, 16)
          redv[s] = redv[s] + tmpv[s]
          return carry
        lax.fori_loop(0, QW // 16, add_j, 0)
      pltpu.sync_copy(
          redv, cnto.at[pl.ds((2 * b + rg_local) * ZPT + qoff, QW)])
      plsc.subcore_barrier()
    pltpu.sync_copy(cnto, cntv)

    for gc in range(5):
      def w_j(j, carry):
        off = gc * 2000 + j * 16
        et16 = etv[pl.ds(off, 16)]
        d16 = dstv[pl.ds(off, 16)]
        pw16 = pwv[pl.ds(off, 16)]
        cidx = et16 * N + d16
        cnt16 = plsc.load_gather(cntv, [cidx])
        sel = jnp.where(et16 >= 4, pw16, jnp.ones((16,), jnp.float32))
        w16 = sel / jnp.maximum(cnt16, 1.0)
        wb[pl.ds(j * 16, 16)] = w16
        return carry
      lax.fori_loop(0, 125, w_j, 0)
      pltpu.sync_copy(wb, w2o.at[t, pl.ds(gc * 2000, 2000)])


_edge_weight_kernel = pl.kernel(
    _edge_weight_body,
    out_type=[
        jax.ShapeDtypeStruct((NS, ET), jnp.float32),
        jax.ShapeDtypeStruct((NS, ET), jnp.int32),
        jax.ShapeDtypeStruct((CNT_PAD,), jnp.float32),
    ],
    mesh=_MESH,
    scratch_types=[
        pltpu.VMEM((ET,), jnp.int32),
        pltpu.VMEM((ET,), jnp.int32),
        pltpu.VMEM((ET,), jnp.int32),
        pltpu.VMEM((ET,), jnp.float32),
        pltpu.VMEM((CNT_PAD,), jnp.float32),
        pltpu.VMEM((2000,), jnp.int32),
        pltpu.VMEM((2000,), jnp.float32),
        pltpu.VMEM((QW,), jnp.float32),
        pltpu.VMEM((QW,), jnp.float32),
        pltpu.VMEM_SHARED((NS, 2, ZPT), jnp.float32),
    ],
    compiler_params=pltpu.CompilerParams(use_tc_tiling_on_sc=False, needs_layout_passes=False),
)


DQ = 64


def _scatter_body(xw4, gv3, dst3, w3, bsum, out, gv, dstv, wv, idxb, rows,
                  zb, bs, acc, sem):
  c = lax.axis_index("c")
  t = lax.axis_index("s")
  pltpu.sync_copy(gv3.at[t], gv)
  pltpu.sync_copy(dst3.at[t], dstv)
  pltpu.sync_copy(w3.at[t], wv)
  pltpu.sync_copy(bsum.at[c], bs)
  base_row = t * RPT

  for f in range(2):
    q = c * 2 + f
    def fill_i(i, carry):
      for fq in range(DQ // 16):
        zb[i, pl.ds(fq * 16, 16)] = bs[f, pl.ds(fq * 16, 16)]
      return carry
    lax.fori_loop(0, 125, fill_i, 0)
    for j in range(RPT // 125):
      pltpu.sync_copy(zb, acc.at[pl.ds(base_row + j * 125, 125)])
    plsc.subcore_barrier()

    def chunk(k, carry):
      def mkidx(g, c2):
        g16 = gv[k, pl.ds(g * 16, 16)]
        idxb[pl.ds(g * 16, 16)] = g16 * 4 + q
        return c2
      lax.fori_loop(0, KC // 16, mkidx, 0)
      pltpu.async_copy(xw4.at[idxb], rows, sem).wait()

      def scale(g, c2):
        w16 = wv[k, pl.ds(g * 16, 16)]
        for j in range(16):
          wvec = jnp.full((16,), w16[j], jnp.float32)
          i = g * 16 + j
          for fq in range(DQ // 16):
            sl = pl.ds(fq * 16, 16)
            rows[i, sl] = rows[i, sl] * wvec
        return c2
      lax.fori_loop(0, KC // 16, scale, 0)
      pltpu.sync_copy(rows, acc.at[dstv.at[k]], add=True)
      return carry
    lax.fori_loop(0, NCH, chunk, 0)
    plsc.subcore_barrier()
    pltpu.sync_copy(acc.at[pl.ds(base_row, RPT)],
                    out.at[c, pl.ds(base_row, RPT), f])


_scatter = pl.kernel(
    _scatter_body,
    out_type=jax.ShapeDtypeStruct((NC, N, 2, DQ), jnp.float32),
    mesh=_MESH,
    scratch_types=[
        pltpu.VMEM((NCH, KC), jnp.int32),
        pltpu.VMEM((NCH, KC), jnp.int32),
        pltpu.VMEM((NCH, KC), jnp.float32),
        pltpu.VMEM((KC,), jnp.int32),
        pltpu.VMEM((KC, DQ), jnp.float32),
        pltpu.VMEM((125, DQ), jnp.float32),
        pltpu.VMEM((2, DQ), jnp.float32),
        pltpu.VMEM_SHARED((N, DQ), jnp.float32),
        pltpu.SemaphoreType.DMA,
    ],
    compiler_params=pltpu.CompilerParams(
        use_tc_tiling_on_sc=False, needs_layout_passes=False),
)


BN_BLK = 1000
NBLK = N // BN_BLK


def _make_mm(do, norm):
  def body(*refs):
    if norm:
      hs_ref, w_ref, s1_ref, s2_ref, g_ref, be_ref, o_ref, hn_ref = refs
      r = pl.program_id(1)

      @pl.when(r == 0)
      def _():
        for c in range(2):
          s1 = s1_ref[c]
          s2 = s2_ref[c]
          mu = s1 * (1.0 / N)
          var = s2 * (1.0 / N) - mu * mu
          scale = lax.rsqrt(var + EPS) * g_ref[c]
          shift = be_ref[c] - mu * scale
          h = hs_ref[c] * scale[None, :] + shift[None, :]
          hn_ref[c] = jnp.where(h >= 0, h, SLOPE * h)
      ha = hn_ref[0]
      hb = hn_ref[1]
    else:
      hs_ref, w_ref, o_ref = refs
      ha = hs_ref[0]
      hb = hs_ref[1]
    o_ref[0] = (
        jnp.dot(ha, w_ref[0, :128, :], preferred_element_type=jnp.float32)
        + jnp.dot(hb, w_ref[0, 128:, :], preferred_element_type=jnp.float32))

  in_specs = [
      pl.BlockSpec((2, BN_BLK, 128), lambda i, r: (0, i, 0)),
      pl.BlockSpec((1, 256, do), lambda i, r: (r, 0, 0)),
  ]
  if norm:
    in_specs += [pl.BlockSpec((2, 128), lambda i, r: (0, 0))] * 4
  return pl.pallas_call(
      functools.partial(body),
      grid=(NBLK, R),
      in_specs=in_specs,
      out_specs=pl.BlockSpec((1, BN_BLK, do), lambda i, r: (r, i, 0)),
      out_shape=jax.ShapeDtypeStruct((R, N, do), jnp.float32),
      scratch_shapes=(
          [pltpu.VMEM((2, BN_BLK, 128), jnp.float32)] if norm else []),
  )


_mm_first = _make_mm(256, norm=False)
_mm_mid = _make_mm(256, norm=True)


def _stats_body(hs_ref, s1_ref, s2_ref):
  i = pl.program_id(0)
  b = hs_ref[...]
  s = jnp.sum(b, axis=1)
  q = jnp.sum(b * b, axis=1)

  @pl.when(i == 0)
  def _():
    s1_ref[...] = s
    s2_ref[...] = q

  @pl.when(i > 0)
  def _():
    s1_ref[...] += s
    s2_ref[...] += q


_stats = pl.pallas_call(
    _stats_body,
    grid=(NBLK,),
    in_specs=[pl.BlockSpec((2, BN_BLK, 128), lambda i: (0, i, 0))],
    out_specs=[pl.BlockSpec((2, 128), lambda i: (0, 0)),
               pl.BlockSpec((2, 128), lambda i: (0, 0))],
    out_shape=[jax.ShapeDtypeStruct((2, 128), jnp.float32),
               jax.ShapeDtypeStruct((2, 128), jnp.float32)],
)


def kernel(x, edge_index, edge_type, pred_weight, W1, b1, W2, b2, W3, b3,
           W4, b4, g1, be1, g2, be2, g3, be3):
  src = edge_index[0]
  dst = edge_index[1]
  et2 = edge_type.reshape(NS, ET)
  src2 = src.reshape(NS, ET)
  dst2 = dst.reshape(NS, ET)
  pw2 = pred_weight.reshape(NS, ET)

  w2, ga, _ = _edge_weight_kernel(et2, src2, dst2, pw2)
  gv3 = ga.reshape(NS, NCH, KC)
  w3 = w2.reshape(NS, NCH, KC)
  dst3 = dst.reshape(NS, NCH, KC)

  hs = jnp.stack([x[:, :128], x[:, 128:]])
  W4p = jnp.concatenate([W4, jnp.zeros((R, 256, 128), jnp.float32)], axis=2)
  b4p = jnp.concatenate([b4, jnp.zeros((R, 128), jnp.float32)], axis=1)
  params = [(W1, None, None, None), (W2, g1, be1, None),
            (W3, g2, be2, None), (W4p, g3, be3, b4p)]
  s1 = s2 = None
  for li, (W, g, be, b) in enumerate(params):
    if li == 0:
      xw = _mm_first(hs, W)
    else:
      xw = _mm_mid(hs, W, s1, s2, g.reshape(2, 128), be.reshape(2, 128))
    xw4 = xw.reshape(R * N * 4, DQ)
    if li == 3:
      bsum = jnp.sum(b, axis=0).reshape(NC, 2, DQ)
    else:
      bsum = jnp.zeros((NC, 2, DQ), jnp.float32)
    hs = _scatter(xw4, gv3, dst3, w3, bsum).reshape(NC, N, 128)
    if li < 3:
      s1, s2 = _stats(hs)
  return hs[0]


# speedup vs baseline: 15.3350x; 1.4779x over previous
import functools

import jax
import jax.numpy as jnp
from jax import lax
from jax.experimental import pallas as pl
from jax.experimental.pallas import tpu as pltpu
import jax.experimental.pallas.tpu_sc as plsc

N = 10000
E = 160000
R = 6
EPS = 1e-5
SLOPE = 0.01

NC = 2
NS = 16
ET = E // NS
KC = 80
NCH = ET // KC
RPT = N // NS
CNT_PAD = 61440
ZPT = CNT_PAD // NS
QW = ZPT // 8

_MESH = plsc.VectorSubcoreMesh(
    core_axis_name="c", subcore_axis_name="s", num_cores=NC, num_subcores=NS)


def _edge_weight_body(et2, src2, dst2, pw2, w2o, gao, cnto,
                      etv, srcv, dstv, pwv, cntv, gab, wb, redv, tmpv,
                      spc):
  c = lax.axis_index("c")
  t = lax.axis_index("s")

  @pl.when(c == 0)
  def _():
    pltpu.sync_copy(et2.at[t], etv)
    pltpu.sync_copy(src2.at[t], srcv)
    pltpu.sync_copy(dst2.at[t], dstv)
    pltpu.sync_copy(pw2.at[t], pwv)

    def zero_j(j, carry):
      cntv[pl.ds(j * 16, 16)] = jnp.zeros((16,), jnp.float32)
      return carry
    lax.fori_loop(0, CNT_PAD // 16, zero_j, 0)

    for gc in range(5):
      def chunk_j(j, carry):
        off = gc * 2000 + j * 16
        et16 = etv[pl.ds(off, 16)]
        d16 = dstv[pl.ds(off, 16)]
        s16 = srcv[pl.ds(off, 16)]
        cidx = et16 * N + d16
        plsc.addupdate_scatter(cntv, [cidx], jnp.ones((16,), jnp.float32))
        gab[pl.ds(j * 16, 16)] = et16 * N + s16
        return carry
      lax.fori_loop(0, 125, chunk_j, 0)
      pltpu.sync_copy(gab, gao.at[t, pl.ds(gc * 2000, 2000)])

    qoff = (t % 8) * QW
    rg_local = t // 8
    for b in range(8):
      for rb in range(2):
        pltpu.sync_copy(cntv.at[pl.ds((2 * b + rb) * ZPT, ZPT)],
                        spc.at[t, rb])
      plsc.subcore_barrier()
      pltpu.sync_copy(spc.at[0, rg_local, pl.ds(qoff, QW)], redv)
      for i in range(1, NS):
        pltpu.sync_copy(spc.at[i, rg_local, pl.ds(qoff, QW)], tmpv)
        def add_j(j, carry):
          s = pl.ds(j * 16, 16)
          redv[s] = redv[s] + tmpv[s]
          return carry
        lax.fori_loop(0, QW // 16, add_j, 0)
      pltpu.sync_copy(
          redv, cnto.at[pl.ds((2 * b + rg_local) * ZPT + qoff, QW)])
      plsc.subcore_barrier()
    pltpu.sync_copy(cnto, cntv)

    for gc in range(5):
      def w_j(j, carry):
        off = gc * 2000 + j * 16
        et16 = etv[pl.ds(off, 16)]
        d16 = dstv[pl.ds(off, 16)]
        pw16 = pwv[pl.ds(off, 16)]
        cidx = et16 * N + d16
        cnt16 = plsc.load_gather(cntv, [cidx])
        sel = jnp.where(et16 >= 4, pw16, jnp.ones((16,), jnp.float32))
        w16 = sel / jnp.maximum(cnt16, 1.0)
        wb[pl.ds(j * 16, 16)] = w16
        return carry
      lax.fori_loop(0, 125, w_j, 0)
      pltpu.sync_copy(wb, w2o.at[t, pl.ds(gc * 2000, 2000)])


_edge_weight_kernel = pl.kernel(
    _edge_weight_body,
    out_type=[
        jax.ShapeDtypeStruct((NS, ET), jnp.float32),
        jax.ShapeDtypeStruct((NS, ET), jnp.int32),
        jax.ShapeDtypeStruct((CNT_PAD,), jnp.float32),
    ],
    mesh=_MESH,
    scratch_types=[
        pltpu.VMEM((ET,), jnp.int32),
        pltpu.VMEM((ET,), jnp.int32),
        pltpu.VMEM((ET,), jnp.int32),
        pltpu.VMEM((ET,), jnp.float32),
        pltpu.VMEM((CNT_PAD,), jnp.float32),
        pltpu.VMEM((2000,), jnp.int32),
        pltpu.VMEM((2000,), jnp.float32),
        pltpu.VMEM((QW,), jnp.float32),
        pltpu.VMEM((QW,), jnp.float32),
        pltpu.VMEM_SHARED((NS, 2, ZPT), jnp.float32),
    ],
    compiler_params=pltpu.CompilerParams(use_tc_tiling_on_sc=False, needs_layout_passes=False),
)


DQ = 64


def _scatter_body(xw4, gv2, dst3, w2, bsum, out, gv, dstv, wv, idxall, rows3,
                  zb, bs, gs0, gs1, gs2, ss0, ss1, ss2, acc):
  c = lax.axis_index("c")
  t = lax.axis_index("s")
  pltpu.sync_copy(gv2.at[t], gv)
  pltpu.sync_copy(dst3.at[t], dstv)
  pltpu.sync_copy(w2.at[t], wv)
  pltpu.sync_copy(bsum.at[c], bs)
  base_row = t * RPT
  gsem = [gs0, gs1, gs2]
  ssem = [ss0, ss1, ss2]

  def g_issue(k, b):
    pltpu.async_copy(xw4.at[idxall.at[pl.ds(k * KC, KC)]], rows3.at[b],
                     gsem[b])

  def g_wait(b):
    pltpu.make_async_copy(xw4.at[idxall.at[pl.ds(0, KC)]], rows3.at[b],
                          gsem[b]).wait()

  def s_issue(k, b):
    pltpu.async_copy(rows3.at[b], acc.at[dstv.at[k]], ssem[b], add=True)

  def s_wait(b):
    pltpu.make_async_copy(rows3.at[b], acc.at[dstv.at[0]], ssem[b]).wait()

  def scale(k, b):
    def grp(g, c2):
      w16 = wv[pl.ds(k * KC + g * 16, 16)]
      for j in range(16):
        wvec = jnp.full((16,), w16[j], jnp.float32)
        i = g * 16 + j
        for fq in range(DQ // 16):
          sl = pl.ds(fq * 16, 16)
          rows3[b, i, sl] = rows3[b, i, sl] * wvec
      return c2
    lax.fori_loop(0, KC // 16, grp, 0)

  def proc(k, b):
    g_wait(b)
    scale(k, b)
    s_issue(k, b)

  for f in range(2):
    q = c * 2 + f

    def mkidx(j, carry):
      s = pl.ds(j * 16, 16)
      idxall[s] = gv[s] * 4 + q
      return carry
    lax.fori_loop(0, ET // 16, mkidx, 0)

    def fill_i(i, carry):
      for fq in range(DQ // 16):
        zb[i, pl.ds(fq * 16, 16)] = bs[f, pl.ds(fq * 16, 16)]
      return carry
    lax.fori_loop(0, 125, fill_i, 0)
    for j in range(RPT // 125):
      pltpu.sync_copy(zb, acc.at[pl.ds(base_row + j * 125, 125)])
    plsc.subcore_barrier()

    g_issue(0, 0)
    g_issue(1, 1)
    proc(0, 0)
    g_issue(2, 2)

    def mbody(m, carry):
      k = 3 * m + 1
      proc(k, 1)
      s_wait(0)
      g_issue(k + 2, 0)
      proc(k + 1, 2)
      s_wait(1)
      g_issue(k + 3, 1)
      proc(k + 2, 0)
      s_wait(2)
      g_issue(k + 4, 2)
      return carry
    lax.fori_loop(0, (NCH - 5) // 3, mbody, 0)

    proc(121, 1)
    s_wait(0)
    g_issue(123, 0)
    proc(122, 2)
    s_wait(1)
    g_issue(124, 1)
    proc(123, 0)
    proc(124, 1)
    s_wait(2)
    s_wait(0)
    s_wait(1)
    plsc.subcore_barrier()
    pltpu.sync_copy(acc.at[pl.ds(base_row, RPT)],
                    out.at[c, pl.ds(base_row, RPT), f])
    if f == 0:
      plsc.subcore_barrier()


_scatter = pl.kernel(
    _scatter_body,
    out_type=jax.ShapeDtypeStruct((NC, N, 2, DQ), jnp.float32),
    mesh=_MESH,
    scratch_types=[
        pltpu.VMEM((ET,), jnp.int32),
        pltpu.VMEM((NCH, KC), jnp.int32),
        pltpu.VMEM((ET,), jnp.float32),
        pltpu.VMEM((ET,), jnp.int32),
        pltpu.VMEM((3, KC, DQ), jnp.float32),
        pltpu.VMEM((125, DQ), jnp.float32),
        pltpu.VMEM((2, DQ), jnp.float32),
        pltpu.SemaphoreType.DMA,
        pltpu.SemaphoreType.DMA,
        pltpu.SemaphoreType.DMA,
        pltpu.SemaphoreType.DMA,
        pltpu.SemaphoreType.DMA,
        pltpu.SemaphoreType.DMA,
        pltpu.VMEM_SHARED((N, DQ), jnp.float32),
    ],
    compiler_params=pltpu.CompilerParams(
        use_tc_tiling_on_sc=False, needs_layout_passes=False),
)


BN_BLK = 1000
NBLK = N // BN_BLK


def _make_mm(do, norm):
  def body(*refs):
    if norm:
      hs_ref, w_ref, s1_ref, s2_ref, g_ref, be_ref, o_ref, hn_ref = refs
      r = pl.program_id(1)

      @pl.when(r == 0)
      def _():
        for c in range(2):
          s1 = s1_ref[c]
          s2 = s2_ref[c]
          mu = s1 * (1.0 / N)
          var = s2 * (1.0 / N) - mu * mu
          scale = lax.rsqrt(var + EPS) * g_ref[c]
          shift = be_ref[c] - mu * scale
          h = hs_ref[c] * scale[None, :] + shift[None, :]
          hn_ref[c] = jnp.where(h >= 0, h, SLOPE * h)
      ha = hn_ref[0]
      hb = hn_ref[1]
    else:
      hs_ref, w_ref, o_ref = refs
      ha = hs_ref[0]
      hb = hs_ref[1]
    o_ref[0] = (
        jnp.dot(ha, w_ref[0, :128, :], preferred_element_type=jnp.float32)
        + jnp.dot(hb, w_ref[0, 128:, :], preferred_element_type=jnp.float32))

  in_specs = [
      pl.BlockSpec((2, BN_BLK, 128), lambda i, r: (0, i, 0)),
      pl.BlockSpec((1, 256, do), lambda i, r: (r, 0, 0)),
  ]
  if norm:
    in_specs += [pl.BlockSpec((2, 128), lambda i, r: (0, 0))] * 4
  return pl.pallas_call(
      functools.partial(body),
      grid=(NBLK, R),
      in_specs=in_specs,
      out_specs=pl.BlockSpec((1, BN_BLK, do), lambda i, r: (r, i, 0)),
      out_shape=jax.ShapeDtypeStruct((R, N, do), jnp.float32),
      scratch_shapes=(
          [pltpu.VMEM((2, BN_BLK, 128), jnp.float32)] if norm else []),
  )


_mm_first = _make_mm(256, norm=False)
_mm_mid = _make_mm(256, norm=True)


def _stats_body(hs_ref, s1_ref, s2_ref):
  i = pl.program_id(0)
  b = hs_ref[...]
  s = jnp.sum(b, axis=1)
  q = jnp.sum(b * b, axis=1)

  @pl.when(i == 0)
  def _():
    s1_ref[...] = s
    s2_ref[...] = q

  @pl.when(i > 0)
  def _():
    s1_ref[...] += s
    s2_ref[...] += q


_stats = pl.pallas_call(
    _stats_body,
    grid=(NBLK,),
    in_specs=[pl.BlockSpec((2, BN_BLK, 128), lambda i: (0, i, 0))],
    out_specs=[pl.BlockSpec((2, 128), lambda i: (0, 0)),
               pl.BlockSpec((2, 128), lambda i: (0, 0))],
    out_shape=[jax.ShapeDtypeStruct((2, 128), jnp.float32),
               jax.ShapeDtypeStruct((2, 128), jnp.float32)],
)


def kernel(x, edge_index, edge_type, pred_weight, W1, b1, W2, b2, W3, b3,
           W4, b4, g1, be1, g2, be2, g3, be3):
  src = edge_index[0]
  dst = edge_index[1]
  et2 = edge_type.reshape(NS, ET)
  src2 = src.reshape(NS, ET)
  dst2 = dst.reshape(NS, ET)
  pw2 = pred_weight.reshape(NS, ET)

  w2, ga, _ = _edge_weight_kernel(et2, src2, dst2, pw2)
  dst3 = dst.reshape(NS, NCH, KC)

  hs = jnp.stack([x[:, :128], x[:, 128:]])
  W4p = jnp.concatenate([W4, jnp.zeros((R, 256, 128), jnp.float32)], axis=2)
  b4p = jnp.concatenate([b4, jnp.zeros((R, 128), jnp.float32)], axis=1)
  params = [(W1, None, None, None), (W2, g1, be1, None),
            (W3, g2, be2, None), (W4p, g3, be3, b4p)]
  s1 = s2 = None
  for li, (W, g, be, b) in enumerate(params):
    if li == 0:
      xw = _mm_first(hs, W)
    else:
      xw = _mm_mid(hs, W, s1, s2, g.reshape(2, 128), be.reshape(2, 128))
    xw4 = xw.reshape(R * N * 4, DQ)
    if li == 3:
      bsum = jnp.sum(b, axis=0).reshape(NC, 2, DQ)
    else:
      bsum = jnp.zeros((NC, 2, DQ), jnp.float32)
    hs = _scatter(xw4, ga, dst3, w2, bsum).reshape(NC, N, 128)
    if li < 3:
      s1, s2 = _stats(hs)
  return hs[0]


# speedup vs baseline: 18.2772x; 1.1919x over previous
import functools

import jax
import jax.numpy as jnp
from jax import lax
from jax.experimental import pallas as pl
from jax.experimental.pallas import tpu as pltpu
import jax.experimental.pallas.tpu_sc as plsc

N = 10000
E = 160000
R = 6
EPS = 1e-5
SLOPE = 0.01

NC = 2
NS = 16
ET = E // NS
KC = 80
NCH = ET // KC
RPT = N // NS
CNT_PAD = 61440
ZPT = CNT_PAD // NS
QW = ZPT // 8

_MESH = plsc.VectorSubcoreMesh(
    core_axis_name="c", subcore_axis_name="s", num_cores=NC, num_subcores=NS)


def _edge_weight_body(et2, src2, dst2, pw2, w2o, gao, cnto,
                      etv, srcv, dstv, pwv, cntv, gab, wb, redv, tmpv,
                      spc):
  c = lax.axis_index("c")
  t = lax.axis_index("s")

  @pl.when(c == 0)
  def _():
    pltpu.sync_copy(et2.at[t], etv)
    pltpu.sync_copy(src2.at[t], srcv)
    pltpu.sync_copy(dst2.at[t], dstv)
    pltpu.sync_copy(pw2.at[t], pwv)

    def zero_j(j, carry):
      cntv[pl.ds(j * 16, 16)] = jnp.zeros((16,), jnp.float32)
      return carry
    lax.fori_loop(0, CNT_PAD // 16, zero_j, 0)

    for gc in range(5):
      def chunk_j(j, carry):
        off = gc * 2000 + j * 16
        et16 = etv[pl.ds(off, 16)]
        d16 = dstv[pl.ds(off, 16)]
        s16 = srcv[pl.ds(off, 16)]
        cidx = et16 * N + d16
        plsc.addupdate_scatter(cntv, [cidx], jnp.ones((16,), jnp.float32))
        gab[pl.ds(j * 16, 16)] = et16 * N + s16
        return carry
      lax.fori_loop(0, 125, chunk_j, 0)
      pltpu.sync_copy(gab, gao.at[t, pl.ds(gc * 2000, 2000)])

    qoff = (t % 8) * QW
    rg_local = t // 8
    for b in range(8):
      for rb in range(2):
        pltpu.sync_copy(cntv.at[pl.ds((2 * b + rb) * ZPT, ZPT)],
                        spc.at[t, rb])
      plsc.subcore_barrier()
      pltpu.sync_copy(spc.at[0, rg_local, pl.ds(qoff, QW)], redv)
      for i in range(1, NS):
        pltpu.sync_copy(spc.at[i, rg_local, pl.ds(qoff, QW)], tmpv)
        def add_j(j, carry):
          s = pl.ds(j * 16, 16)
          redv[s] = redv[s] + tmpv[s]
          return carry
        lax.fori_loop(0, QW // 16, add_j, 0)
      pltpu.sync_copy(
          redv, cnto.at[pl.ds((2 * b + rg_local) * ZPT + qoff, QW)])
      plsc.subcore_barrier()
    pltpu.sync_copy(cnto, cntv)

    for gc in range(5):
      def w_j(j, carry):
        off = gc * 2000 + j * 16
        et16 = etv[pl.ds(off, 16)]
        d16 = dstv[pl.ds(off, 16)]
        pw16 = pwv[pl.ds(off, 16)]
        cidx = et16 * N + d16
        cnt16 = plsc.load_gather(cntv, [cidx])
        sel = jnp.where(et16 >= 4, pw16, jnp.ones((16,), jnp.float32))
        w16 = sel / jnp.maximum(cnt16, 1.0)
        wb[pl.ds(j * 16, 16)] = w16
        return carry
      lax.fori_loop(0, 125, w_j, 0)
      pltpu.sync_copy(wb, w2o.at[t, pl.ds(gc * 2000, 2000)])


_edge_weight_kernel = pl.kernel(
    _edge_weight_body,
    out_type=[
        jax.ShapeDtypeStruct((NS, ET), jnp.float32),
        jax.ShapeDtypeStruct((NS, ET), jnp.int32),
        jax.ShapeDtypeStruct((CNT_PAD,), jnp.float32),
    ],
    mesh=_MESH,
    scratch_types=[
        pltpu.VMEM((ET,), jnp.int32),
        pltpu.VMEM((ET,), jnp.int32),
        pltpu.VMEM((ET,), jnp.int32),
        pltpu.VMEM((ET,), jnp.float32),
        pltpu.VMEM((CNT_PAD,), jnp.float32),
        pltpu.VMEM((2000,), jnp.int32),
        pltpu.VMEM((2000,), jnp.float32),
        pltpu.VMEM((QW,), jnp.float32),
        pltpu.VMEM((QW,), jnp.float32),
        pltpu.VMEM_SHARED((NS, 2, ZPT), jnp.float32),
    ],
    compiler_params=pltpu.CompilerParams(use_tc_tiling_on_sc=False, needs_layout_passes=False),
)


DQ = 64


def _scatter_body(xw4, gv2, dst3, w2, bsum, out, gv, dstv, wv, idxall, rows3,
                  zb, bs, gs0, gs1, gs2, ss0, ss1, ss2, acc):
  c = lax.axis_index("c")
  t = lax.axis_index("s")
  pltpu.sync_copy(gv2.at[t], gv)
  pltpu.sync_copy(dst3.at[t], dstv)
  pltpu.sync_copy(w2.at[t], wv)
  pltpu.sync_copy(bsum.at[c], bs)
  base_row = t * RPT
  gsem = [gs0, gs1, gs2]
  ssem = [ss0, ss1, ss2]

  def g_issue(k, b):
    pltpu.async_copy(xw4.at[idxall.at[pl.ds(k * KC, KC)]], rows3.at[b],
                     gsem[b])

  def g_wait(b):
    pltpu.make_async_copy(xw4.at[idxall.at[pl.ds(0, KC)]], rows3.at[b],
                          gsem[b]).wait()

  def s_issue(k, b):
    pltpu.async_copy(rows3.at[b], acc.at[dstv.at[k]], ssem[b], add=True)

  def s_wait(b):
    pltpu.make_async_copy(rows3.at[b], acc.at[dstv.at[0]], ssem[b]).wait()

  def scale(k, b):
    def grp(g, c2):
      w16 = wv[pl.ds(k * KC + g * 16, 16)]
      for j in range(16):
        wvec = jnp.full((16,), w16[j], jnp.float32)
        i = g * 16 + j
        for fq in range(DQ // 16):
          sl = pl.ds(fq * 16, 16)
          rows3[b, i, sl] = rows3[b, i, sl] * wvec
      return c2
    lax.fori_loop(0, KC // 16, grp, 0)

  def proc(k, b):
    g_wait(b)
    scale(k, b)
    s_issue(k, b)

  for f in range(2):
    q = c * 2 + f

    def mkidx(j, carry):
      s = pl.ds(j * 16, 16)
      idxall[s] = gv[s] * 4 + q
      return carry
    lax.fori_loop(0, ET // 16, mkidx, 0)

    def fill_i(i, carry):
      for fq in range(DQ // 16):
        zb[i, pl.ds(fq * 16, 16)] = bs[f, pl.ds(fq * 16, 16)]
      return carry
    lax.fori_loop(0, 125, fill_i, 0)
    for j in range(RPT // 125):
      pltpu.sync_copy(zb, acc.at[pl.ds(base_row + j * 125, 125)])
    plsc.subcore_barrier()

    g_issue(0, 0)
    g_issue(1, 1)
    proc(0, 0)
    g_issue(2, 2)

    def mbody(m, carry):
      k = 3 * m + 1
      proc(k, 1)
      s_wait(0)
      g_issue(k + 2, 0)
      proc(k + 1, 2)
      s_wait(1)
      g_issue(k + 3, 1)
      proc(k + 2, 0)
      s_wait(2)
      g_issue(k + 4, 2)
      return carry
    lax.fori_loop(0, (NCH - 5) // 3, mbody, 0)

    proc(121, 1)
    s_wait(0)
    g_issue(123, 0)
    proc(122, 2)
    s_wait(1)
    g_issue(124, 1)
    proc(123, 0)
    proc(124, 1)
    s_wait(2)
    s_wait(0)
    s_wait(1)
    plsc.subcore_barrier()
    pltpu.sync_copy(acc.at[pl.ds(base_row, RPT)],
                    out.at[pl.ds(base_row, RPT), pl.ds(q * DQ, DQ)])
    if f == 0:
      plsc.subcore_barrier()


_scatter = pl.kernel(
    _scatter_body,
    out_type=jax.ShapeDtypeStruct((N, 4 * DQ), jnp.float32),
    mesh=_MESH,
    scratch_types=[
        pltpu.VMEM((ET,), jnp.int32),
        pltpu.VMEM((NCH, KC), jnp.int32),
        pltpu.VMEM((ET,), jnp.float32),
        pltpu.VMEM((ET,), jnp.int32),
        pltpu.VMEM((3, KC, DQ), jnp.float32),
        pltpu.VMEM((125, DQ), jnp.float32),
        pltpu.VMEM((2, DQ), jnp.float32),
        pltpu.SemaphoreType.DMA,
        pltpu.SemaphoreType.DMA,
        pltpu.SemaphoreType.DMA,
        pltpu.SemaphoreType.DMA,
        pltpu.SemaphoreType.DMA,
        pltpu.SemaphoreType.DMA,
        pltpu.VMEM_SHARED((N, DQ), jnp.float32),
    ],
    compiler_params=pltpu.CompilerParams(
        use_tc_tiling_on_sc=False, needs_layout_passes=False),
)


BN_BLK = 1000
NBLK = N // BN_BLK


def _make_mm(do, norm):
  def body(*refs):
    if norm:
      hs_ref, w_ref, s1_ref, s2_ref, g_ref, be_ref, o_ref, hn_ref = refs
      r = pl.program_id(1)

      @pl.when(r == 0)
      def _():
        mu = s1_ref[0] * (1.0 / N)
        var = s2_ref[0] * (1.0 / N) - mu * mu
        scale = lax.rsqrt(var + EPS) * g_ref[0]
        shift = be_ref[0] - mu * scale
        h = hs_ref[...] * scale[None, :] + shift[None, :]
        hn_ref[...] = jnp.where(h >= 0, h, SLOPE * h)
      ha = hn_ref[...]
    else:
      hs_ref, w_ref, o_ref = refs
      ha = hs_ref[...]
    o_ref[0] = jnp.dot(ha, w_ref[0], preferred_element_type=jnp.float32)

  in_specs = [
      pl.BlockSpec((BN_BLK, 256), lambda i, r: (i, 0)),
      pl.BlockSpec((1, 256, do), lambda i, r: (r, 0, 0)),
  ]
  if norm:
    in_specs += [pl.BlockSpec((1, 256), lambda i, r: (0, 0))] * 4
  return pl.pallas_call(
      functools.partial(body),
      grid=(NBLK, R),
      in_specs=in_specs,
      out_specs=pl.BlockSpec((1, BN_BLK, do), lambda i, r: (r, i, 0)),
      out_shape=jax.ShapeDtypeStruct((R, N, do), jnp.float32),
      scratch_shapes=(
          [pltpu.VMEM((BN_BLK, 256), jnp.float32)] if norm else []),
  )


_mm_first = _make_mm(256, norm=False)
_mm_mid = _make_mm(256, norm=True)


def _stats_body(hs_ref, s1_ref, s2_ref):
  i = pl.program_id(0)
  b = hs_ref[...]
  s = jnp.sum(b, axis=0)
  q = jnp.sum(b * b, axis=0)

  @pl.when(i == 0)
  def _():
    s1_ref[0] = s
    s2_ref[0] = q

  @pl.when(i > 0)
  def _():
    s1_ref[0] += s
    s2_ref[0] += q


_stats = pl.pallas_call(
    _stats_body,
    grid=(NBLK,),
    in_specs=[pl.BlockSpec((BN_BLK, 256), lambda i: (i, 0))],
    out_specs=[pl.BlockSpec((1, 256), lambda i: (0, 0)),
               pl.BlockSpec((1, 256), lambda i: (0, 0))],
    out_shape=[jax.ShapeDtypeStruct((1, 256), jnp.float32),
               jax.ShapeDtypeStruct((1, 256), jnp.float32)],
)


def kernel(x, edge_index, edge_type, pred_weight, W1, b1, W2, b2, W3, b3,
           W4, b4, g1, be1, g2, be2, g3, be3):
  src = edge_index[0]
  dst = edge_index[1]
  et2 = edge_type.reshape(NS, ET)
  src2 = src.reshape(NS, ET)
  dst2 = dst.reshape(NS, ET)
  pw2 = pred_weight.reshape(NS, ET)

  w2, ga, _ = _edge_weight_kernel(et2, src2, dst2, pw2)
  dst3 = dst.reshape(NS, NCH, KC)

  hs = x
  W4p = jnp.concatenate([W4, jnp.zeros((R, 256, 128), jnp.float32)], axis=2)
  b4p = jnp.concatenate([b4, jnp.zeros((R, 128), jnp.float32)], axis=1)
  params = [(W1, None, None, None), (W2, g1, be1, None),
            (W3, g2, be2, None), (W4p, g3, be3, b4p)]
  s1 = s2 = None
  for li, (W, g, be, b) in enumerate(params):
    if li == 0:
      xw = _mm_first(hs, W)
    else:
      xw = _mm_mid(hs, W, s1, s2, g.reshape(1, 256), be.reshape(1, 256))
    xw4 = xw.reshape(R * N * 4, DQ)
    if li == 3:
      bsum = jnp.sum(b, axis=0).reshape(NC, 2, DQ)
    else:
      bsum = jnp.zeros((NC, 2, DQ), jnp.float32)
    hs = _scatter(xw4, ga, dst3, w2, bsum)
    if li < 3:
      s1, s2 = _stats(hs)
  return hs[:, :128]
